# baseline (speedup 1.0000x reference)
import functools

import jax
import jax.numpy as jnp
from jax import lax
from jax.experimental import pallas as pl
from jax.experimental.pallas import tpu as pltpu
from jax.experimental.pallas import tpu_sc as plsc

DOC_LEN = 16384
EMB = 64
NUM_WORDS = 1000000
PAD_WORDS = 1 << 20
NUM_CP = 100000
NUM_TILES = 32
ROWS_PER_TILE = DOC_LEN // NUM_TILES
PER_TILE_WORDS = PAD_WORDS // 16
BLK = 16384
GRID = (NUM_WORDS + BLK - 1) // BLK


def _sc_counts(x0):
    mesh = plsc.VectorSubcoreMesh(core_axis_name="c", subcore_axis_name="s")

    @functools.partial(
        pl.kernel,
        out_type=jax.ShapeDtypeStruct((2 * PAD_WORDS,), jnp.float32),
        mesh=mesh,
        scratch_types=[
            pltpu.VMEM((4, 128), jnp.int32),
            pltpu.VMEM((128,), jnp.float32),
            pltpu.VMEM((PER_TILE_WORDS,), jnp.float32),
            pltpu.VMEM_SHARED((NUM_WORDS,), jnp.float32),
        ],
    )
    def k(x0_hbm, cnt_hbm, idx_v, ones_v, z_v, cnt_s):
        cid = lax.axis_index("c")
        sid = lax.axis_index("s")
        wid = sid * 2 + cid
        base = wid * ROWS_PER_TILE

        for j in range(4):
            pltpu.sync_copy(x0_hbm.at[pl.ds(base + j * 128, 128)],
                            idx_v.at[j])

        one = jnp.full((16,), 1.0, jnp.float32)
        for j in range(8):
            ones_v[pl.ds(16 * j, 16)] = one

        zero = jnp.zeros((16,), jnp.float32)

        def zb(i, c):
            for j in range(16):
                z_v[pl.ds(i * 256 + j * 16, 16)] = zero
            return c

        lax.fori_loop(0, PER_TILE_WORDS // 256, zb, 0)

        tail = NUM_WORDS - 15 * PER_TILE_WORDS

        @pl.when(sid < 15)
        def _():
            pltpu.sync_copy(z_v, cnt_s.at[pl.ds(sid * PER_TILE_WORDS,
                                                PER_TILE_WORDS)])

        @pl.when(sid == 15)
        def _():
            pltpu.sync_copy(z_v.at[pl.ds(0, tail)],
                            cnt_s.at[pl.ds(15 * PER_TILE_WORDS, tail)])

        plsc.subcore_barrier()
        for j in range(4):
            pltpu.sync_copy(ones_v, cnt_s.at[idx_v.at[j]], add=True)
        plsc.subcore_barrier()

        cbase = cid * PAD_WORDS

        @pl.when(sid < 15)
        def _():
            pltpu.sync_copy(
                cnt_s.at[pl.ds(sid * PER_TILE_WORDS, PER_TILE_WORDS)], z_v)
            pltpu.sync_copy(
                z_v,
                cnt_hbm.at[pl.ds(cbase + sid * PER_TILE_WORDS,
                                 PER_TILE_WORDS)])

        @pl.when(sid == 15)
        def _():
            pltpu.sync_copy(
                z_v.at[pl.ds(0, PAD_WORDS - NUM_WORDS)],
                cnt_hbm.at[pl.ds(cbase + NUM_WORDS,
                                 PAD_WORDS - NUM_WORDS)])
            pltpu.sync_copy(
                cnt_s.at[pl.ds(15 * PER_TILE_WORDS, tail)],
                z_v.at[pl.ds(0, tail)])
            pltpu.sync_copy(
                z_v.at[pl.ds(0, tail)],
                cnt_hbm.at[pl.ds(cbase + 15 * PER_TILE_WORDS, tail)])

    return k(x0)


def _tc_stage(x1, counts, emb1_t, emb2_t, x2, W1, b1, W2, b2):
    def body(x1_ref, tbl_ref, c0_ref, c1_ref, x2_ref, w1_ref, b1_ref,
             w2_ref, b2_ref, e2_hbm, o_ref, acc_ref, e2_v, sem):
        i = pl.program_id(0)

        @pl.when(i == 0)
        def _():
            acc_ref[...] = jnp.zeros_like(acc_ref)
            pltpu.async_copy(e2_hbm, e2_v, sem)

        c = (c0_ref[...] + c1_ref[...]).reshape(1, BLK)
        acc_ref[...] += lax.dot_general(
            c, tbl_ref[...], (((1,), (1,)), ((), ())),
            preferred_element_type=jnp.float32)

        @pl.when(i == GRID - 1)
        def _():
            s = acc_ref[...] * (1.0 / DOC_LEN)
            y1 = lax.dot_general(s, w1_ref[...], (((1,), (1,)), ((), ())),
                                 preferred_element_type=jnp.float32)
            y1 = y1 + b1_ref[...]
            y3 = jax.nn.sigmoid(
                lax.dot_general(x2_ref[...], w2_ref[...],
                                (((1,), (1,)), ((), ())),
                                preferred_element_type=jnp.float32)
                + b2_ref[...])
            pltpu.make_async_copy(e2_hbm, e2_v, sem).wait()
            oh = (lax.broadcasted_iota(jnp.int32, (1, NUM_CP), 1)
                  == x1_ref[0]).astype(jnp.float32)
            y2 = lax.dot_general(oh, e2_v[...], (((1,), (1,)), ((), ())),
                                 preferred_element_type=jnp.float32)
            t = y2 + y3
            o_ref[...] = jax.nn.sigmoid(jnp.sum(y1 * t, axis=1,
                                                keepdims=True))

    grid_spec = pltpu.PrefetchScalarGridSpec(
        num_scalar_prefetch=1,
        grid=(GRID,),
        in_specs=[
            pl.BlockSpec((EMB, BLK), lambda i, x1r: (0, i)),
            pl.BlockSpec((BLK,), lambda i, x1r: (i,)),
            pl.BlockSpec((BLK,), lambda i, x1r: (PAD_WORDS // BLK + i,)),
            pl.BlockSpec((1, EMB), lambda i, x1r: (0, 0)),
            pl.BlockSpec((EMB, EMB), lambda i, x1r: (0, 0)),
            pl.BlockSpec((1, EMB), lambda i, x1r: (0, 0)),
            pl.BlockSpec((EMB, EMB), lambda i, x1r: (0, 0)),
            pl.BlockSpec((1, EMB), lambda i, x1r: (0, 0)),
            pl.BlockSpec(memory_space=pl.ANY),
        ],
        out_specs=pl.BlockSpec((1, 1), lambda i, x1r: (0, 0)),
        scratch_shapes=[
            pltpu.VMEM((1, EMB), jnp.float32),
            pltpu.VMEM((EMB, NUM_CP), jnp.float32),
            pltpu.SemaphoreType.DMA,
        ],
    )
    return pl.pallas_call(
        body,
        grid_spec=grid_spec,
        out_shape=jax.ShapeDtypeStruct((1, 1), jnp.float32),
        compiler_params=pltpu.CompilerParams(
            dimension_semantics=("arbitrary",)),
    )(x1, emb1_t, counts, counts, x2, W1, b1, W2, b2, emb2_t)


def kernel(x0, x1, x2, emb1, emb2, W1, b1, W2, b2):
    counts = _sc_counts(x0)
    out = _tc_stage(x1.astype(jnp.int32), counts, emb1.T, emb2.T,
                    x2.reshape(1, EMB), W1, b1.reshape(1, EMB),
                    W2, b2.reshape(1, EMB))
    return out.reshape(())


# speedup vs baseline: 1.1114x; 1.1114x over previous
import functools

import jax
import jax.numpy as jnp
from jax import lax
from jax.experimental import pallas as pl
from jax.experimental.pallas import tpu as pltpu
from jax.experimental.pallas import tpu_sc as plsc

DOC_LEN = 16384
EMB = 64
NUM_WORDS = 1000000
PAD_WORDS = 1 << 20
NUM_CP = 100000
NUM_TILES = 32
ROWS_PER_TILE = DOC_LEN // NUM_TILES
PER_TILE_WORDS = PAD_WORDS // 16
BLK = 32768
GRID = (NUM_WORDS + BLK - 1) // BLK


def _sc_counts(x0):
    mesh = plsc.VectorSubcoreMesh(core_axis_name="c", subcore_axis_name="s")

    @functools.partial(
        pl.kernel,
        out_type=jax.ShapeDtypeStruct((2 * PAD_WORDS,), jnp.float32),
        mesh=mesh,
        scratch_types=[
            pltpu.VMEM((4, 128), jnp.int32),
            pltpu.VMEM((128,), jnp.float32),
            pltpu.VMEM((PER_TILE_WORDS,), jnp.float32),
            pltpu.VMEM_SHARED((NUM_WORDS,), jnp.float32),
        ],
    )
    def k(x0_hbm, cnt_hbm, idx_v, ones_v, z_v, cnt_s):
        cid = lax.axis_index("c")
        sid = lax.axis_index("s")
        wid = sid * 2 + cid
        base = wid * ROWS_PER_TILE

        for j in range(4):
            pltpu.sync_copy(x0_hbm.at[pl.ds(base + j * 128, 128)],
                            idx_v.at[j])

        one = jnp.full((16,), 1.0, jnp.float32)
        for j in range(8):
            ones_v[pl.ds(16 * j, 16)] = one

        zero = jnp.zeros((16,), jnp.float32)

        def zb(i, c):
            for j in range(16):
                z_v[pl.ds(i * 256 + j * 16, 16)] = zero
            return c

        lax.fori_loop(0, PER_TILE_WORDS // 256, zb, 0)

        tail = NUM_WORDS - 15 * PER_TILE_WORDS

        @pl.when(sid < 15)
        def _():
            pltpu.sync_copy(z_v, cnt_s.at[pl.ds(sid * PER_TILE_WORDS,
                                                PER_TILE_WORDS)])

        @pl.when(sid == 15)
        def _():
            pltpu.sync_copy(z_v.at[pl.ds(0, tail)],
                            cnt_s.at[pl.ds(15 * PER_TILE_WORDS, tail)])

        plsc.subcore_barrier()
        for j in range(4):
            pltpu.sync_copy(ones_v, cnt_s.at[idx_v.at[j]], add=True)
        plsc.subcore_barrier()

        cbase = cid * PAD_WORDS

        @pl.when(sid < 15)
        def _():
            pltpu.sync_copy(
                cnt_s.at[pl.ds(sid * PER_TILE_WORDS, PER_TILE_WORDS)], z_v)
            pltpu.sync_copy(
                z_v,
                cnt_hbm.at[pl.ds(cbase + sid * PER_TILE_WORDS,
                                 PER_TILE_WORDS)])

        @pl.when(sid == 15)
        def _():
            pltpu.sync_copy(
                z_v.at[pl.ds(0, PAD_WORDS - NUM_WORDS)],
                cnt_hbm.at[pl.ds(cbase + NUM_WORDS,
                                 PAD_WORDS - NUM_WORDS)])
            pltpu.sync_copy(
                cnt_s.at[pl.ds(15 * PER_TILE_WORDS, tail)],
                z_v.at[pl.ds(0, tail)])
            pltpu.sync_copy(
                z_v.at[pl.ds(0, tail)],
                cnt_hbm.at[pl.ds(cbase + 15 * PER_TILE_WORDS, tail)])

    return k(x0)


def _tc_stage(x1, counts, emb1_t, emb2_t, x2, W1, b1, W2, b2):
    def body(x1_ref, tbl_ref, c0_ref, c1_ref, x2_ref, w1_ref, b1_ref,
             w2_ref, b2_ref, e2_hbm, o_ref, acc_ref, e2_v, sem):
        i = pl.program_id(0)

        @pl.when(i == 0)
        def _():
            acc_ref[...] = jnp.zeros_like(acc_ref)
            pltpu.async_copy(e2_hbm, e2_v, sem)

        c = (c0_ref[...] + c1_ref[...]).reshape(1, BLK)
        acc_ref[...] += lax.dot_general(
            c, tbl_ref[...], (((1,), (1,)), ((), ())),
            preferred_element_type=jnp.float32)

        @pl.when(i == GRID - 1)
        def _():
            s = acc_ref[...] * (1.0 / DOC_LEN)
            y1 = lax.dot_general(s, w1_ref[...], (((1,), (1,)), ((), ())),
                                 preferred_element_type=jnp.float32)
            y1 = y1 + b1_ref[...]
            y3 = jax.nn.sigmoid(
                lax.dot_general(x2_ref[...], w2_ref[...],
                                (((1,), (1,)), ((), ())),
                                preferred_element_type=jnp.float32)
                + b2_ref[...])
            pltpu.make_async_copy(e2_hbm, e2_v, sem).wait()
            oh = (lax.broadcasted_iota(jnp.int32, (1, NUM_CP), 1)
                  == x1_ref[0]).astype(jnp.float32)
            y2 = lax.dot_general(oh, e2_v[...], (((1,), (1,)), ((), ())),
                                 preferred_element_type=jnp.float32)
            t = y2 + y3
            o_ref[...] = jax.nn.sigmoid(jnp.sum(y1 * t, axis=1,
                                                keepdims=True))

    grid_spec = pltpu.PrefetchScalarGridSpec(
        num_scalar_prefetch=1,
        grid=(GRID,),
        in_specs=[
            pl.BlockSpec((EMB, BLK), lambda i, x1r: (0, i)),
            pl.BlockSpec((BLK,), lambda i, x1r: (i,)),
            pl.BlockSpec((BLK,), lambda i, x1r: (PAD_WORDS // BLK + i,)),
            pl.BlockSpec((1, EMB), lambda i, x1r: (0, 0)),
            pl.BlockSpec((EMB, EMB), lambda i, x1r: (0, 0)),
            pl.BlockSpec((1, EMB), lambda i, x1r: (0, 0)),
            pl.BlockSpec((EMB, EMB), lambda i, x1r: (0, 0)),
            pl.BlockSpec((1, EMB), lambda i, x1r: (0, 0)),
            pl.BlockSpec(memory_space=pl.ANY),
        ],
        out_specs=pl.BlockSpec((1, 1), lambda i, x1r: (0, 0)),
        scratch_shapes=[
            pltpu.VMEM((1, EMB), jnp.float32),
            pltpu.VMEM((EMB, NUM_CP), jnp.float32),
            pltpu.SemaphoreType.DMA,
        ],
    )
    return pl.pallas_call(
        body,
        grid_spec=grid_spec,
        out_shape=jax.ShapeDtypeStruct((1, 1), jnp.float32),
        compiler_params=pltpu.CompilerParams(
            dimension_semantics=("arbitrary",)),
    )(x1, emb1_t, counts, counts, x2, W1, b1, W2, b2, emb2_t)


def kernel(x0, x1, x2, emb1, emb2, W1, b1, W2, b2):
    counts = _sc_counts(x0)
    out = _tc_stage(x1.astype(jnp.int32), counts, emb1.T, emb2.T,
                    x2.reshape(1, EMB), W1, b1.reshape(1, EMB),
                    W2, b2.reshape(1, EMB))
    return out.reshape(())


# speedup vs baseline: 1.1307x; 1.0174x over previous
import functools

import jax
import jax.numpy as jnp
from jax import lax
from jax.experimental import pallas as pl
from jax.experimental.pallas import tpu as pltpu
from jax.experimental.pallas import tpu_sc as plsc

DOC_LEN = 16384
EMB = 64
NUM_WORDS = 1000000
PAD_WORDS = 1 << 20
NUM_CP = 100000
NUM_TILES = 32
ROWS_PER_TILE = DOC_LEN // NUM_TILES
PER_TILE_WORDS = PAD_WORDS // 16
BLK = 32768
GRID = (NUM_WORDS + BLK - 1) // BLK


def _sc_counts(x0):
    mesh = plsc.VectorSubcoreMesh(core_axis_name="c", subcore_axis_name="s")

    @functools.partial(
        pl.kernel,
        out_type=jax.ShapeDtypeStruct((2 * PAD_WORDS,), jnp.float32),
        mesh=mesh,
        scratch_types=[
            pltpu.VMEM((4, 128), jnp.int32),
            pltpu.VMEM((128,), jnp.float32),
            pltpu.VMEM((PER_TILE_WORDS,), jnp.float32),
            pltpu.VMEM_SHARED((NUM_WORDS,), jnp.float32),
            pltpu.SemaphoreType.DMA,
        ],
    )
    def k(x0_hbm, cnt_hbm, idx_v, ones_v, z_v, cnt_s, zsem):
        cid = lax.axis_index("c")
        sid = lax.axis_index("s")
        wid = sid * 2 + cid
        base = wid * ROWS_PER_TILE

        for j in range(4):
            pltpu.sync_copy(x0_hbm.at[pl.ds(base + j * 128, 128)],
                            idx_v.at[j])

        one = jnp.full((16,), 1.0, jnp.float32)
        for j in range(8):
            ones_v[pl.ds(16 * j, 16)] = one

        zero = jnp.zeros((16,), jnp.float32)

        def zb(i, c):
            for j in range(16):
                z_v[pl.ds(i * 256 + j * 16, 16)] = zero
            return c

        lax.fori_loop(0, 16, zb, 0)
        z4 = z_v.at[pl.ds(0, 4096)]

        tail = NUM_WORDS - 15 * PER_TILE_WORDS

        @pl.when(sid < 15)
        def _():
            for k in range(16):
                pltpu.async_copy(
                    z4, cnt_s.at[pl.ds(sid * PER_TILE_WORDS + k * 4096,
                                       4096)], zsem)
            for k in range(16):
                pltpu.make_async_copy(
                    z4, cnt_s.at[pl.ds(sid * PER_TILE_WORDS + k * 4096,
                                       4096)], zsem).wait()

        @pl.when(sid == 15)
        def _():
            for k in range(4):
                pltpu.async_copy(
                    z4, cnt_s.at[pl.ds(15 * PER_TILE_WORDS + k * 4096,
                                       4096)], zsem)
            pltpu.async_copy(
                z_v.at[pl.ds(0, tail - 16384)],
                cnt_s.at[pl.ds(15 * PER_TILE_WORDS + 16384, tail - 16384)],
                zsem)
            for k in range(4):
                pltpu.make_async_copy(
                    z4, cnt_s.at[pl.ds(15 * PER_TILE_WORDS + k * 4096,
                                       4096)], zsem).wait()
            pltpu.make_async_copy(
                z_v.at[pl.ds(0, tail - 16384)],
                cnt_s.at[pl.ds(15 * PER_TILE_WORDS + 16384, tail - 16384)],
                zsem).wait()

        plsc.subcore_barrier()
        for j in range(4):
            pltpu.sync_copy(ones_v, cnt_s.at[idx_v.at[j]], add=True)
        plsc.subcore_barrier()

        cbase = cid * PAD_WORDS

        @pl.when(sid < 15)
        def _():
            pltpu.sync_copy(
                cnt_s.at[pl.ds(sid * PER_TILE_WORDS, PER_TILE_WORDS)], z_v)
            pltpu.sync_copy(
                z_v,
                cnt_hbm.at[pl.ds(cbase + sid * PER_TILE_WORDS,
                                 PER_TILE_WORDS)])

        @pl.when(sid == 15)
        def _():
            for k in range(12):
                sz = 4096 if k < 11 else PAD_WORDS - NUM_WORDS - 11 * 4096
                pltpu.async_copy(
                    z_v.at[pl.ds(0, sz)],
                    cnt_hbm.at[pl.ds(cbase + NUM_WORDS + k * 4096, sz)],
                    zsem)
            for k in range(12):
                sz = 4096 if k < 11 else PAD_WORDS - NUM_WORDS - 11 * 4096
                pltpu.make_async_copy(
                    z_v.at[pl.ds(0, sz)],
                    cnt_hbm.at[pl.ds(cbase + NUM_WORDS + k * 4096, sz)],
                    zsem).wait()
            pltpu.sync_copy(
                cnt_s.at[pl.ds(15 * PER_TILE_WORDS, tail)],
                z_v.at[pl.ds(0, tail)])
            pltpu.sync_copy(
                z_v.at[pl.ds(0, tail)],
                cnt_hbm.at[pl.ds(cbase + 15 * PER_TILE_WORDS, tail)])

    return k(x0)


def _tc_stage(x1, counts, emb1_t, emb2_t, x2, W1, b1, W2, b2):
    def body(x1_ref, tbl_ref, c0_ref, c1_ref, x2_ref, w1_ref, b1_ref,
             w2_ref, b2_ref, e2_hbm, o_ref, acc_ref, e2_v, sem):
        i = pl.program_id(0)

        @pl.when(i == 0)
        def _():
            acc_ref[...] = jnp.zeros_like(acc_ref)
            pltpu.async_copy(e2_hbm, e2_v, sem)

        c = (c0_ref[...] + c1_ref[...]).reshape(1, BLK)
        acc_ref[...] += lax.dot_general(
            c, tbl_ref[...], (((1,), (1,)), ((), ())),
            preferred_element_type=jnp.float32)

        @pl.when(i == GRID - 1)
        def _():
            s = acc_ref[...] * (1.0 / DOC_LEN)
            y1 = lax.dot_general(s, w1_ref[...], (((1,), (1,)), ((), ())),
                                 preferred_element_type=jnp.float32)
            y1 = y1 + b1_ref[...]
            y3 = jax.nn.sigmoid(
                lax.dot_general(x2_ref[...], w2_ref[...],
                                (((1,), (1,)), ((), ())),
                                preferred_element_type=jnp.float32)
                + b2_ref[...])
            pltpu.make_async_copy(e2_hbm, e2_v, sem).wait()
            oh = (lax.broadcasted_iota(jnp.int32, (1, NUM_CP), 1)
                  == x1_ref[0]).astype(jnp.float32)
            y2 = lax.dot_general(oh, e2_v[...], (((1,), (1,)), ((), ())),
                                 preferred_element_type=jnp.float32)
            t = y2 + y3
            o_ref[...] = jax.nn.sigmoid(jnp.sum(y1 * t, axis=1,
                                                keepdims=True))

    grid_spec = pltpu.PrefetchScalarGridSpec(
        num_scalar_prefetch=1,
        grid=(GRID,),
        in_specs=[
            pl.BlockSpec((EMB, BLK), lambda i, x1r: (0, i)),
            pl.BlockSpec((BLK,), lambda i, x1r: (i,)),
            pl.BlockSpec((BLK,), lambda i, x1r: (PAD_WORDS // BLK + i,)),
            pl.BlockSpec((1, EMB), lambda i, x1r: (0, 0)),
            pl.BlockSpec((EMB, EMB), lambda i, x1r: (0, 0)),
            pl.BlockSpec((1, EMB), lambda i, x1r: (0, 0)),
            pl.BlockSpec((EMB, EMB), lambda i, x1r: (0, 0)),
            pl.BlockSpec((1, EMB), lambda i, x1r: (0, 0)),
            pl.BlockSpec(memory_space=pl.ANY),
        ],
        out_specs=pl.BlockSpec((1, 1), lambda i, x1r: (0, 0)),
        scratch_shapes=[
            pltpu.VMEM((1, EMB), jnp.float32),
            pltpu.VMEM((EMB, NUM_CP), jnp.float32),
            pltpu.SemaphoreType.DMA,
        ],
    )
    return pl.pallas_call(
        body,
        grid_spec=grid_spec,
        out_shape=jax.ShapeDtypeStruct((1, 1), jnp.float32),
        compiler_params=pltpu.CompilerParams(
            dimension_semantics=("arbitrary",)),
    )(x1, emb1_t, counts, counts, x2, W1, b1, W2, b2, emb2_t)


def kernel(x0, x1, x2, emb1, emb2, W1, b1, W2, b2):
    counts = _sc_counts(x0)
    out = _tc_stage(x1.astype(jnp.int32), counts, emb1.T, emb2.T,
                    x2.reshape(1, EMB), W1, b1.reshape(1, EMB),
                    W2, b2.reshape(1, EMB))
    return out.reshape(())
